# Initial kernel scaffold; baseline (speedup 1.0000x reference)
import jax
import jax.numpy as jnp
from jax.experimental import pallas as pl


def kernel(h, diff_cart, dist_sq, edge_src, edge_dst, t_emb_edges, t_emb_nodes, edge_w1, edge_b1, edge_w2, edge_b2, coord_w1, coord_b1, coord_w2, node_w1, node_b1, node_w2, node_b2):
    raise NotImplementedError("write your pallas kernel here")


# speedup vs baseline: 2.2727x; 2.2727x over previous
import functools

import jax
import jax.numpy as jnp
from jax import lax
from jax.experimental import pallas as pl
from jax.experimental.pallas import tpu as pltpu
from jax.experimental.pallas import tpu_sc as plsc

_NC = 2
_NS = 16
_KG = 80
_KS = 80


def _pre_body(h_ref, tn_ref, w1a_ref, w1b_ref, b1_ref, nw1a_ref, nw1c_ref,
              nb1_ref, a_out, b_out, np_out):
    h = h_ref[...]
    a_out[...] = h @ w1a_ref[...] + b1_ref[...]
    b_out[...] = h @ w1b_ref[...]
    np_out[...] = h @ nw1a_ref[...] + tn_ref[...] @ nw1c_ref[...] + nb1_ref[...]


def _edge_body(ga_ref, gb_ref, dist_ref, temb_ref, diff_ref,
               w1d_ref, w1c_ref, w2_ref, b2_ref, cw1_ref, cb1_ref, cw2_ref,
               mij_out, cs_out):
    pre = (ga_ref[...] + gb_ref[...] + dist_ref[...] * w1c_ref[...]
           + temb_ref[...] @ w1d_ref[...])
    m = pre * jax.nn.sigmoid(pre)
    m2 = m @ w2_ref[...] + b2_ref[...]
    mij = m2 * jax.nn.sigmoid(m2)
    ch = mij @ cw1_ref[...] + cb1_ref[...]
    ch = ch * jax.nn.sigmoid(ch)
    cw = ch @ cw2_ref[...]
    mij_out[...] = mij
    cs_out[...] = diff_ref[...] * cw


def _node_body(h_ref, np_ref, mi0_ref, mi1_ref, cp0_ref, cp1_ref,
               nw1b_ref, nw2_ref, nb2_ref, hu_out, cu_out):
    mi = mi0_ref[...] + mi1_ref[...]
    t = np_ref[...] + mi @ nw1b_ref[...]
    t = t * jax.nn.sigmoid(t)
    hu_out[...] = h_ref[...] + (t @ nw2_ref[...] + nb2_ref[...])
    cu_out[...] = (cp0_ref[...] + cp1_ref[...])[:, :3]


def _make_gather(E, D):
    per = E // (_NC * _NS)
    nch = per // _KG
    mesh = plsc.VectorSubcoreMesh(core_axis_name="c", subcore_axis_name="s")

    @functools.partial(
        pl.kernel,
        out_type=[jax.ShapeDtypeStruct((E, D), jnp.float32),
                  jax.ShapeDtypeStruct((E, D), jnp.float32)],
        mesh=mesh,
        scratch_types=[pltpu.VMEM((_KG,), jnp.int32),
                       pltpu.VMEM((_KG, D), jnp.float32),
                       pltpu.SemaphoreType.DMA],
    )
    def gather_k(a_hbm, b_hbm, src_hbm, dst_hbm, ga_hbm, gb_hbm,
                 idx_v, rows_v, sem):
        wid = lax.axis_index("s") * _NC + lax.axis_index("c")
        base = pl.multiple_of(wid * per, 8)
        for ci in range(nch):
            off = base + ci * _KG
            pltpu.sync_copy(src_hbm.at[pl.ds(off, _KG)], idx_v)
            pltpu.async_copy(a_hbm.at[idx_v], rows_v, sem).wait()
            pltpu.sync_copy(rows_v, ga_hbm.at[pl.ds(off, _KG)])
            pltpu.sync_copy(dst_hbm.at[pl.ds(off, _KG)], idx_v)
            pltpu.async_copy(b_hbm.at[idx_v], rows_v, sem).wait()
            pltpu.sync_copy(rows_v, gb_hbm.at[pl.ds(off, _KG)])

    return gather_k


def _scatter_rows(N, K, W, rows_lo, mij_hbm, idx_hbm, z_hbm, out_hbm,
                  sh, idx_v, rows_v, nch, per):
    c = lax.axis_index("c")
    s = lax.axis_index("s")
    wid = s * _NC + c
    tail = N - rows_lo * _NS

    def row_chunks():
        chunks = []
        done = 0
        while done < rows_lo:
            step = min(K, rows_lo - done)
            chunks.append((done, step))
            done += step
        return chunks

    r0 = pl.multiple_of(s * rows_lo, 8)
    pltpu.sync_copy(z_hbm, rows_v)
    for done, step in row_chunks():
        pltpu.sync_copy(rows_v.at[pl.ds(0, step)], sh.at[pl.ds(r0 + done, step)])

    @pl.when(s == _NS - 1)
    def _zero_tail():
        pltpu.sync_copy(rows_v.at[pl.ds(0, tail)], sh.at[pl.ds(N - tail, tail)])

    plsc.subcore_barrier()
    base = pl.multiple_of(wid * per, 8)

    def chunk_body(ci, _):
        off = pl.multiple_of(base + ci * K, 8)
        pltpu.sync_copy(idx_hbm.at[pl.ds(off, K)], idx_v)
        pltpu.sync_copy(mij_hbm.at[pl.ds(off, K)], rows_v)
        pltpu.sync_copy(rows_v, sh.at[idx_v], add=True)
        return 0

    lax.fori_loop(0, nch, chunk_body, 0)
    plsc.subcore_barrier()
    for done, step in row_chunks():
        pltpu.sync_copy(sh.at[pl.ds(r0 + done, step)], rows_v.at[pl.ds(0, step)])
        pltpu.sync_copy(rows_v.at[pl.ds(0, step)],
                        out_hbm.at[c, pl.ds(r0 + done, step)])

    @pl.when(s == _NS - 1)
    def _write_tail():
        pltpu.sync_copy(sh.at[pl.ds(N - tail, tail)], rows_v.at[pl.ds(0, tail)])
        pltpu.sync_copy(rows_v.at[pl.ds(0, tail)],
                        out_hbm.at[c, pl.ds(N - tail, tail)])


def _make_scatter(E, N, W, K):
    per = E // (_NC * _NS)
    nch = per // K
    rows_lo = (N // _NS) // 8 * 8
    mesh = plsc.VectorSubcoreMesh(core_axis_name="c", subcore_axis_name="s")

    @functools.partial(
        pl.kernel,
        out_type=jax.ShapeDtypeStruct((_NC, N, W), jnp.float32),
        mesh=mesh,
        scratch_types=[pltpu.VMEM_SHARED((N, W), jnp.float32),
                       pltpu.VMEM((K,), jnp.int32),
                       pltpu.VMEM((K, W), jnp.float32)],
    )
    def scatter_k(val_hbm, idx_hbm, z_hbm, out_hbm, sh, idx_v, rows_v):
        _scatter_rows(N, K, W, rows_lo, val_hbm, idx_hbm, z_hbm, out_hbm,
                      sh, idx_v, rows_v, nch, per)

    return scatter_k


def kernel(h, diff_cart, dist_sq, edge_src, edge_dst, t_emb_edges, t_emb_nodes,
           edge_w1, edge_b1, edge_w2, edge_b2,
           coord_w1, coord_b1, coord_w2,
           node_w1, node_b1, node_w2, node_b2):
    N, D = h.shape
    E = edge_src.shape[0]
    T = t_emb_edges.shape[1]
    f32 = jnp.float32

    src = edge_src.astype(jnp.int32)
    dst = edge_dst.astype(jnp.int32)
    w1a = edge_w1[:D]
    w1b = edge_w1[D:2 * D]
    w1c = edge_w1[2 * D:2 * D + 1]
    w1d = edge_w1[2 * D + 1:]
    nw1a = node_w1[:D]
    nw1b = node_w1[D:2 * D]
    nw1c = node_w1[2 * D:]
    b1 = edge_b1.reshape(1, D)
    b2 = edge_b2.reshape(1, D)
    cb1 = coord_b1.reshape(1, D)
    nb1 = node_b1.reshape(1, D)
    nb2 = node_b2.reshape(1, D)
    diffp = jnp.concatenate([diff_cart, jnp.zeros((E, D - 3), f32)], axis=1)

    BN = 2000
    BE = 2000
    gn = N // BN
    ge = E // BE

    def row_spec(b, w):
        return pl.BlockSpec((b, w), lambda i: (i, 0))

    def full_spec(shape):
        return pl.BlockSpec(shape, lambda i: tuple(0 for _ in shape))

    a_tab, b_tab, node_pre = pl.pallas_call(
        _pre_body,
        grid=(gn,),
        in_specs=[row_spec(BN, D), row_spec(BN, T),
                  full_spec((D, D)), full_spec((D, D)), full_spec((1, D)),
                  full_spec((D, D)), full_spec((T, D)), full_spec((1, D))],
        out_specs=[row_spec(BN, D)] * 3,
        out_shape=[jax.ShapeDtypeStruct((N, D), f32)] * 3,
    )(h, t_emb_nodes, w1a, w1b, b1, nw1a, nw1c, nb1)

    ga, gb = _make_gather(E, D)(a_tab, b_tab, src, dst)

    m_ij, coord_shift = pl.pallas_call(
        _edge_body,
        grid=(ge,),
        in_specs=[row_spec(BE, D), row_spec(BE, D), row_spec(BE, 1),
                  row_spec(BE, T), row_spec(BE, D),
                  full_spec((T, D)), full_spec((1, D)), full_spec((D, D)),
                  full_spec((1, D)), full_spec((D, D)), full_spec((1, D)),
                  full_spec((D, 1))],
        out_specs=[row_spec(BE, D), row_spec(BE, D)],
        out_shape=[jax.ShapeDtypeStruct((E, D), f32),
                   jax.ShapeDtypeStruct((E, D), f32)],
    )(ga, gb, dist_sq, t_emb_edges, diffp,
      w1d, w1c, edge_w2, b2, coord_w1, cb1, coord_w2)

    z = jnp.zeros((_KS, D), f32)
    mi_part = _make_scatter(E, N, D, _KS)(m_ij, dst, z)
    cp_part = _make_scatter(E, N, D, _KS)(coord_shift, src, z)

    h_update, coord_update = pl.pallas_call(
        _node_body,
        grid=(gn,),
        in_specs=[row_spec(BN, D), row_spec(BN, D), row_spec(BN, D),
                  row_spec(BN, D), row_spec(BN, D), row_spec(BN, D),
                  full_spec((D, D)), full_spec((D, D)), full_spec((1, D))],
        out_specs=[row_spec(BN, D), row_spec(BN, 3)],
        out_shape=[jax.ShapeDtypeStruct((N, D), f32),
                   jax.ShapeDtypeStruct((N, 3), f32)],
    )(h, node_pre, mi_part[0], mi_part[1], cp_part[0], cp_part[1],
      nw1b, node_w2, nb2)

    return (h_update, coord_update)


# speedup vs baseline: 3.0244x; 1.3307x over previous
import functools

import jax
import jax.numpy as jnp
from jax import lax
from jax.experimental import pallas as pl
from jax.experimental.pallas import tpu as pltpu
from jax.experimental.pallas import tpu_sc as plsc

_NC = 2
_NS = 16
_KG = 200
_KS = 80


def _pre_body(h_ref, tn_ref, w1a_ref, w1b_ref, b1_ref, nw1a_ref, nw1c_ref,
              nb1_ref, a_out, b_out, np_out):
    h = h_ref[...]
    a_out[...] = h @ w1a_ref[...] + b1_ref[...]
    b_out[...] = h @ w1b_ref[...]
    np_out[...] = h @ nw1a_ref[...] + tn_ref[...] @ nw1c_ref[...] + nb1_ref[...]


def _edge_body(ga_ref, gb_ref, dist_ref, temb_ref, diff_ref,
               w1d_ref, w1c_ref, w2_ref, b2_ref, cw1_ref, cb1_ref, cw2_ref,
               mij_out, cs_out):
    pre = (ga_ref[...] + gb_ref[...] + dist_ref[...] * w1c_ref[...]
           + temb_ref[...] @ w1d_ref[...])
    m = pre * jax.nn.sigmoid(pre)
    m2 = m @ w2_ref[...] + b2_ref[...]
    mij = m2 * jax.nn.sigmoid(m2)
    ch = mij @ cw1_ref[...] + cb1_ref[...]
    ch = ch * jax.nn.sigmoid(ch)
    cw = ch @ cw2_ref[...]
    mij_out[...] = mij
    pad = jnp.zeros((cw.shape[0], cs_out.shape[-1] - 4), cw.dtype)
    cs_out[...] = jnp.concatenate([diff_ref[...] * cw, pad], axis=-1)


def _node_body(h_ref, np_ref, mi0_ref, mi1_ref, cp0_ref, cp1_ref,
               nw1b_ref, nw2_ref, nb2_ref, hu_out, cu_out):
    mi = mi0_ref[...] + mi1_ref[...]
    t = np_ref[...] + mi @ nw1b_ref[...]
    t = t * jax.nn.sigmoid(t)
    hu_out[...] = h_ref[...] + (t @ nw2_ref[...] + nb2_ref[...])
    cu_out[...] = (cp0_ref[...] + cp1_ref[...])[:, :3]


def _make_gather(E, D):
    per = E // (_NC * _NS)
    nch = per // _KG
    mesh = plsc.VectorSubcoreMesh(core_axis_name="c", subcore_axis_name="s")

    @functools.partial(
        pl.kernel,
        out_type=[jax.ShapeDtypeStruct((E, D), jnp.float32),
                  jax.ShapeDtypeStruct((E, D), jnp.float32)],
        mesh=mesh,
        scratch_types=[pltpu.VMEM((_KG,), jnp.int32),
                       pltpu.VMEM((_KG,), jnp.int32),
                       pltpu.VMEM((_KG, D), jnp.float32),
                       pltpu.VMEM((_KG, D), jnp.float32),
                       pltpu.SemaphoreType.DMA,
                       pltpu.SemaphoreType.DMA,
                       pltpu.SemaphoreType.DMA,
                       pltpu.SemaphoreType.DMA,
                       pltpu.SemaphoreType.DMA,
                       pltpu.SemaphoreType.DMA],
    )
    def gather_k(a_hbm, b_hbm, src_hbm, dst_hbm, ga_hbm, gb_hbm,
                 idxa_v, idxb_v, rowsa_v, rowsb_v,
                 sia, sib, sga, sgb, swa, swb):
        wid = lax.axis_index("s") * _NC + lax.axis_index("c")
        base = pl.multiple_of(wid * per, 8)

        def body(ci, _):
            @pl.when(ci > 0)
            def _drain():
                pltpu.make_async_copy(
                    rowsa_v, ga_hbm.at[pl.ds(base, _KG)], swa).wait()
                pltpu.make_async_copy(
                    rowsb_v, gb_hbm.at[pl.ds(base, _KG)], swb).wait()

            off = pl.multiple_of(base + ci * _KG, 8)
            ia = pltpu.async_copy(src_hbm.at[pl.ds(off, _KG)], idxa_v, sia)
            ib = pltpu.async_copy(dst_hbm.at[pl.ds(off, _KG)], idxb_v, sib)
            ia.wait()
            ca = pltpu.async_copy(a_hbm.at[idxa_v], rowsa_v, sga)
            ib.wait()
            cb = pltpu.async_copy(b_hbm.at[idxb_v], rowsb_v, sgb)
            ca.wait()
            pltpu.async_copy(rowsa_v, ga_hbm.at[pl.ds(off, _KG)], swa)
            cb.wait()
            pltpu.async_copy(rowsb_v, gb_hbm.at[pl.ds(off, _KG)], swb)
            return 0

        lax.fori_loop(0, nch, body, 0)
        pltpu.make_async_copy(rowsa_v, ga_hbm.at[pl.ds(base, _KG)], swa).wait()
        pltpu.make_async_copy(rowsb_v, gb_hbm.at[pl.ds(base, _KG)], swb).wait()

    return gather_k


def _make_scatter(E, N, W, K):
    per = E // (_NC * _NS)
    nch = per // K
    npair = nch // 2
    odd = nch % 2
    rows_lo = (N // _NS) // 8 * 8
    mesh = plsc.VectorSubcoreMesh(core_axis_name="c", subcore_axis_name="s")

    @functools.partial(
        pl.kernel,
        out_type=jax.ShapeDtypeStruct((_NC, N, W), jnp.float32),
        mesh=mesh,
        scratch_types=[pltpu.VMEM_SHARED((N, W), jnp.float32),
                       pltpu.VMEM((K,), jnp.int32),
                       pltpu.VMEM((K,), jnp.int32),
                       pltpu.VMEM((K, W), jnp.float32),
                       pltpu.VMEM((K, W), jnp.float32),
                       pltpu.SemaphoreType.DMA,
                       pltpu.SemaphoreType.DMA,
                       pltpu.SemaphoreType.DMA,
                       pltpu.SemaphoreType.DMA,
                       pltpu.SemaphoreType.DMA,
                       pltpu.SemaphoreType.DMA],
    )
    def scatter_k(val_hbm, idx_hbm, z_hbm, out_hbm, sh,
                  idx0_v, idx1_v, rows0_v, rows1_v,
                  si0, si1, sv0, sv1, ss0, ss1):
        c = lax.axis_index("c")
        s = lax.axis_index("s")
        wid = s * _NC + c
        tail = N - rows_lo * _NS

        def row_chunks():
            chunks = []
            done = 0
            while done < rows_lo:
                step = min(K, rows_lo - done)
                chunks.append((done, step))
                done += step
            return chunks

        r0 = pl.multiple_of(s * rows_lo, 8)
        pltpu.sync_copy(z_hbm, rows0_v)
        for done, step in row_chunks():
            pltpu.sync_copy(rows0_v.at[pl.ds(0, step)],
                            sh.at[pl.ds(r0 + done, step)])

        @pl.when(s == _NS - 1)
        def _zero_tail():
            pltpu.sync_copy(rows0_v.at[pl.ds(0, tail)],
                            sh.at[pl.ds(N - tail, tail)])

        plsc.subcore_barrier()
        base = pl.multiple_of(wid * per, 8)

        def drain_pair():
            pltpu.make_async_copy(rows0_v, sh.at[idx0_v], ss0).wait()
            pltpu.make_async_copy(rows1_v, sh.at[idx1_v], ss1).wait()

        def body(k, _):
            @pl.when(k > 0)
            def _drain():
                drain_pair()

            off0 = pl.multiple_of(base + (2 * k) * K, 8)
            off1 = pl.multiple_of(base + (2 * k + 1) * K, 8)
            i0 = pltpu.async_copy(idx_hbm.at[pl.ds(off0, K)], idx0_v, si0)
            v0 = pltpu.async_copy(val_hbm.at[pl.ds(off0, K)], rows0_v, sv0)
            i1 = pltpu.async_copy(idx_hbm.at[pl.ds(off1, K)], idx1_v, si1)
            v1 = pltpu.async_copy(val_hbm.at[pl.ds(off1, K)], rows1_v, sv1)
            i0.wait()
            v0.wait()
            pltpu.async_copy(rows0_v, sh.at[idx0_v], ss0, add=True)
            i1.wait()
            v1.wait()
            pltpu.async_copy(rows1_v, sh.at[idx1_v], ss1, add=True)
            return 0

        lax.fori_loop(0, npair, body, 0)
        drain_pair()
        if odd:
            lastoff = pl.multiple_of(base + (nch - 1) * K, 8)
            pltpu.sync_copy(idx_hbm.at[pl.ds(lastoff, K)], idx0_v)
            pltpu.sync_copy(val_hbm.at[pl.ds(lastoff, K)], rows0_v)
            pltpu.sync_copy(rows0_v, sh.at[idx0_v], add=True)
        plsc.subcore_barrier()
        for done, step in row_chunks():
            pltpu.sync_copy(sh.at[pl.ds(r0 + done, step)],
                            rows0_v.at[pl.ds(0, step)])
            pltpu.sync_copy(rows0_v.at[pl.ds(0, step)],
                            out_hbm.at[c, pl.ds(r0 + done, step)])

        @pl.when(s == _NS - 1)
        def _write_tail():
            pltpu.sync_copy(sh.at[pl.ds(N - tail, tail)],
                            rows0_v.at[pl.ds(0, tail)])
            pltpu.sync_copy(rows0_v.at[pl.ds(0, tail)],
                            out_hbm.at[c, pl.ds(N - tail, tail)])

    return scatter_k


def kernel(h, diff_cart, dist_sq, edge_src, edge_dst, t_emb_edges, t_emb_nodes,
           edge_w1, edge_b1, edge_w2, edge_b2,
           coord_w1, coord_b1, coord_w2,
           node_w1, node_b1, node_w2, node_b2):
    N, D = h.shape
    E = edge_src.shape[0]
    T = t_emb_edges.shape[1]
    f32 = jnp.float32

    src = edge_src.astype(jnp.int32)
    dst = edge_dst.astype(jnp.int32)
    w1a = edge_w1[:D]
    w1b = edge_w1[D:2 * D]
    w1c = edge_w1[2 * D:2 * D + 1]
    w1d = edge_w1[2 * D + 1:]
    nw1a = node_w1[:D]
    nw1b = node_w1[D:2 * D]
    nw1c = node_w1[2 * D:]
    b1 = edge_b1.reshape(1, D)
    b2 = edge_b2.reshape(1, D)
    cb1 = coord_b1.reshape(1, D)
    nb1 = node_b1.reshape(1, D)
    nb2 = node_b2.reshape(1, D)
    diff4 = jnp.concatenate([diff_cart, jnp.zeros((E, 1), f32)], axis=1)

    BN = 2000
    BE = 2000
    gn = N // BN
    ge = E // BE

    def row_spec(b, w):
        return pl.BlockSpec((b, w), lambda i: (i, 0))

    def full_spec(shape):
        return pl.BlockSpec(shape, lambda i: tuple(0 for _ in shape))

    a_tab, b_tab, node_pre = pl.pallas_call(
        _pre_body,
        grid=(gn,),
        in_specs=[row_spec(BN, D), row_spec(BN, T),
                  full_spec((D, D)), full_spec((D, D)), full_spec((1, D)),
                  full_spec((D, D)), full_spec((T, D)), full_spec((1, D))],
        out_specs=[row_spec(BN, D)] * 3,
        out_shape=[jax.ShapeDtypeStruct((N, D), f32)] * 3,
    )(h, t_emb_nodes, w1a, w1b, b1, nw1a, nw1c, nb1)

    ga, gb = _make_gather(E, D)(a_tab, b_tab, src, dst)

    m_ij, coord_shift = pl.pallas_call(
        _edge_body,
        grid=(ge,),
        in_specs=[row_spec(BE, D), row_spec(BE, D), row_spec(BE, 1),
                  row_spec(BE, T), row_spec(BE, 4),
                  full_spec((T, D)), full_spec((1, D)), full_spec((D, D)),
                  full_spec((1, D)), full_spec((D, D)), full_spec((1, D)),
                  full_spec((D, 1))],
        out_specs=[row_spec(BE, D), row_spec(BE, D)],
        out_shape=[jax.ShapeDtypeStruct((E, D), f32),
                   jax.ShapeDtypeStruct((E, D), f32)],
    )(ga, gb, dist_sq, t_emb_edges, diff4,
      w1d, w1c, edge_w2, b2, coord_w1, cb1, coord_w2)

    z = jnp.zeros((_KS, D), f32)
    mi_part = _make_scatter(E, N, D, _KS)(m_ij, dst, z)
    cp_part = _make_scatter(E, N, D, _KS)(coord_shift, src, z)

    h_update, coord_update = pl.pallas_call(
        _node_body,
        grid=(gn,),
        in_specs=[row_spec(BN, D), row_spec(BN, D), row_spec(BN, D),
                  row_spec(BN, D), row_spec(BN, D), row_spec(BN, D),
                  full_spec((D, D)), full_spec((D, D)), full_spec((1, D))],
        out_specs=[row_spec(BN, D), row_spec(BN, 3)],
        out_shape=[jax.ShapeDtypeStruct((N, D), f32),
                   jax.ShapeDtypeStruct((N, 3), f32)],
    )(h, node_pre, mi_part[0], mi_part[1], cp_part[0], cp_part[1],
      nw1b, node_w2, nb2)

    return (h_update, coord_update)
